# Initial kernel scaffold; baseline (speedup 1.0000x reference)
import jax
import jax.numpy as jnp
from jax.experimental import pallas as pl


def kernel(x, edge_index, W_self0, W_neigh0, b0, W_self1, W_neigh1, b1, W_self2, W_neigh2, b2):
    raise NotImplementedError("write your pallas kernel here")


# speedup vs baseline: 4.0095x; 4.0095x over previous
import functools

import jax
import jax.numpy as jnp
from jax import lax
from jax.experimental import pallas as pl
from jax.experimental.pallas import tpu as pltpu
from jax.experimental.pallas import tpu_sc as plsc

NC = 2
NS = 16
CPY = 64


def _transform_body(h_ref, ws_ref, wn_ref, b_ref, hs_ref, ht_ref):
    h = h_ref[...]
    hs_ref[...] = jnp.dot(h, ws_ref[...], preferred_element_type=jnp.float32) + b_ref[...]
    ht_ref[...] = jnp.dot(h, wn_ref[...], preferred_element_type=jnp.float32)


def _transform(h, W_self, W_neigh, b, blk=512):
    n, d = h.shape
    dout = W_self.shape[1]
    return pl.pallas_call(
        _transform_body,
        grid=(pl.cdiv(n, blk),),
        in_specs=[
            pl.BlockSpec((blk, d), lambda i: (i, 0)),
            pl.BlockSpec((d, dout), lambda i: (0, 0)),
            pl.BlockSpec((d, dout), lambda i: (0, 0)),
            pl.BlockSpec((1, dout), lambda i: (0, 0)),
        ],
        out_specs=[
            pl.BlockSpec((blk, dout), lambda i: (i, 0)),
            pl.BlockSpec((blk, dout), lambda i: (i, 0)),
        ],
        out_shape=[jax.ShapeDtypeStruct((n, dout), jnp.float32)] * 2,
    )(h, W_self, W_neigh, b.reshape(1, dout))


def _combine_transform_body(hs_ref, agg_ref, deg_ref, ws_ref, wn_ref, b_ref,
                            hs2_ref, ht2_ref):
    agg = agg_ref[0] + agg_ref[1]
    deg = deg_ref[0] + deg_ref[1]
    inv = 1.0 / jnp.maximum(deg, 1.0)
    h = jnp.maximum(hs_ref[...] + agg * inv, 0.0)
    hs2_ref[...] = jnp.dot(h, ws_ref[...], preferred_element_type=jnp.float32) + b_ref[...]
    ht2_ref[...] = jnp.dot(h, wn_ref[...], preferred_element_type=jnp.float32)


def _combine_transform(hs, aggp, degp, W_self, W_neigh, b, blk=512):
    n, d = hs.shape
    dout = W_self.shape[1]
    return pl.pallas_call(
        _combine_transform_body,
        grid=(pl.cdiv(n, blk),),
        in_specs=[
            pl.BlockSpec((blk, d), lambda i: (i, 0)),
            pl.BlockSpec((NC, blk, d), lambda i: (0, i, 0)),
            pl.BlockSpec((NC, blk, d), lambda i: (0, i, 0)),
            pl.BlockSpec((d, dout), lambda i: (0, 0)),
            pl.BlockSpec((d, dout), lambda i: (0, 0)),
            pl.BlockSpec((1, dout), lambda i: (0, 0)),
        ],
        out_specs=[
            pl.BlockSpec((blk, dout), lambda i: (i, 0)),
            pl.BlockSpec((blk, dout), lambda i: (i, 0)),
        ],
        out_shape=[jax.ShapeDtypeStruct((n, dout), jnp.float32)] * 2,
    )(hs, aggp, degp, W_self, W_neigh, b.reshape(1, dout))


def _combine_keep_body(hs_ref, agg_ref, deg_ref, ws_ref, b_ref,
                       hs2_ref, h_ref):
    agg = agg_ref[0] + agg_ref[1]
    deg = deg_ref[0] + deg_ref[1]
    inv = 1.0 / jnp.maximum(deg, 1.0)
    h = jnp.maximum(hs_ref[...] + agg * inv, 0.0)
    hs2_ref[...] = jnp.dot(h, ws_ref[...], preferred_element_type=jnp.float32) + b_ref[...]
    h_ref[...] = h


def _combine_keep(hs, aggp, degp, W_self, b, blk=512):
    n, d = hs.shape
    dout = W_self.shape[1]
    return pl.pallas_call(
        _combine_keep_body,
        grid=(pl.cdiv(n, blk),),
        in_specs=[
            pl.BlockSpec((blk, d), lambda i: (i, 0)),
            pl.BlockSpec((NC, blk, d), lambda i: (0, i, 0)),
            pl.BlockSpec((NC, blk, d), lambda i: (0, i, 0)),
            pl.BlockSpec((d, dout), lambda i: (0, 0)),
            pl.BlockSpec((1, dout), lambda i: (0, 0)),
        ],
        out_specs=[
            pl.BlockSpec((blk, dout), lambda i: (i, 0)),
            pl.BlockSpec((blk, d), lambda i: (i, 0)),
        ],
        out_shape=[jax.ShapeDtypeStruct((n, dout), jnp.float32),
                   jax.ShapeDtypeStruct((n, d), jnp.float32)],
    )(hs, aggp, degp, W_self, b.reshape(1, dout))


def _final_body(hs_ref, agg_ref, deg_ref, wn_ref, out_ref):
    agg = agg_ref[0] + agg_ref[1]
    deg = deg_ref[0] + deg_ref[1]
    inv = 1.0 / jnp.maximum(deg, 1.0)
    out_ref[...] = hs_ref[...] + jnp.dot(
        agg * inv, wn_ref[...], preferred_element_type=jnp.float32)


def _final(hs, aggp, degp, W_neigh, blk=512):
    n, d = hs.shape
    dagg = aggp.shape[2]
    return pl.pallas_call(
        _final_body,
        grid=(pl.cdiv(n, blk),),
        in_specs=[
            pl.BlockSpec((blk, d), lambda i: (i, 0)),
            pl.BlockSpec((NC, blk, dagg), lambda i: (0, i, 0)),
            pl.BlockSpec((NC, blk, dagg), lambda i: (0, i, 0)),
            pl.BlockSpec((dagg, d), lambda i: (0, 0)),
        ],
        out_specs=pl.BlockSpec((blk, d), lambda i: (i, 0)),
        out_shape=jax.ShapeDtypeStruct((n, d), jnp.float32),
    )(hs, aggp, degp, W_neigh)


def _pick_chunk(epw):
    for ch in (128, 120, 112, 104, 96, 88, 80, 72, 64, 56, 48, 40, 32, 24, 16, 8):
        if epw % ch == 0:
            return ch
    return 8


@functools.lru_cache(maxsize=None)
def _make_agg(n, e, d):
    nw = NC * NS
    epw = e // nw
    ch = _pick_chunk(epw)
    steps = epw // ch
    rpt = ((n + NS - 1) // NS + CPY - 1) // CPY * CPY
    npad = rpt * NS
    ncopy = rpt // CPY

    mesh = plsc.VectorSubcoreMesh(core_axis_name="c", subcore_axis_name="s",
                                  num_cores=NC, num_subcores=NS)

    @functools.partial(
        pl.kernel, mesh=mesh,
        out_type=[jax.ShapeDtypeStruct((NC, npad, d), jnp.float32)],
        scratch_types=[
            pltpu.VMEM_SHARED((npad, d), jnp.float32),
            pltpu.VMEM((ch,), jnp.int32),
            pltpu.VMEM((ch,), jnp.int32),
            pltpu.VMEM((ch, d), jnp.float32),
            pltpu.SemaphoreType.DMA,
        ])
    def kern(table, srcv, dstv, zeros_nd, agg_out, acc, idx_s, idx_d, rows,
             sem):
        c = lax.axis_index("c")
        s = lax.axis_index("s")
        wid = c * NS + s
        r0 = s * rpt
        pltpu.sync_copy(zeros_nd.at[pl.ds(0, CPY)], rows.at[pl.ds(0, CPY)])
        for k in range(ncopy):
            pltpu.sync_copy(rows.at[pl.ds(0, CPY)],
                            acc.at[pl.ds(r0 + k * CPY, CPY)])
        plsc.subcore_barrier()
        base = wid * epw

        def step(j, carry):
            off = base + j * ch
            pltpu.sync_copy(srcv.at[pl.ds(off, ch)], idx_s)
            pltpu.sync_copy(dstv.at[pl.ds(off, ch)], idx_d)
            pltpu.async_copy(table.at[idx_s], rows, sem).wait()
            pltpu.sync_copy(rows, acc.at[idx_d], add=True)
            return carry

        lax.fori_loop(0, steps, step, 0)
        plsc.subcore_barrier()
        for k in range(ncopy):
            pltpu.sync_copy(acc.at[pl.ds(r0 + k * CPY, CPY)],
                            rows.at[pl.ds(0, CPY)])
            pltpu.sync_copy(rows.at[pl.ds(0, CPY)],
                            agg_out.at[c].at[pl.ds(r0 + k * CPY, CPY)])

    def run(table, src, dst):
        zeros_nd = jnp.zeros((npad, d), jnp.float32)
        return kern(table, src, dst, zeros_nd)[0]

    return run


def kernel(x, edge_index, W_self0, W_neigh0, b0, W_self1, W_neigh1, b1,
           W_self2, W_neigh2, b2):
    src = edge_index[0]
    dst = edge_index[1]
    n, _ = x.shape
    e = src.shape[0]
    d_hid = W_self0.shape[1]

    agg_hid = _make_agg(n, e, d_hid)

    degp = agg_hid(jnp.ones((n, d_hid), jnp.float32), src, dst)

    hs0, ht0 = _transform(x, W_self0, W_neigh0, b0)
    aggp0 = agg_hid(ht0, src, dst)
    hs1, ht1 = _combine_transform(hs0, aggp0, degp, W_self1, W_neigh1, b1)
    aggp1 = agg_hid(ht1, src, dst)
    hs2, h2 = _combine_keep(hs1, aggp1, degp, W_self2, b2)
    aggp2 = agg_hid(h2, src, dst)
    return _final(hs2, aggp2, degp, W_neigh2)


# speedup vs baseline: 8.3965x; 2.0942x over previous
import functools

import jax
import jax.numpy as jnp
from jax import lax
from jax.experimental import pallas as pl
from jax.experimental.pallas import tpu as pltpu
from jax.experimental.pallas import tpu_sc as plsc

NC = 2
NS = 16
CPY = 64


def _transform_body(h_ref, ws_ref, wn_ref, b_ref, hs_ref, ht_ref):
    h = h_ref[...]
    hs_ref[...] = jnp.dot(h, ws_ref[...], preferred_element_type=jnp.float32) + b_ref[...]
    ht_ref[...] = jnp.dot(h, wn_ref[...], preferred_element_type=jnp.float32)


def _transform(h, W_self, W_neigh, b, blk=512):
    n, d = h.shape
    dout = W_self.shape[1]
    return pl.pallas_call(
        _transform_body,
        grid=(pl.cdiv(n, blk),),
        in_specs=[
            pl.BlockSpec((blk, d), lambda i: (i, 0)),
            pl.BlockSpec((d, dout), lambda i: (0, 0)),
            pl.BlockSpec((d, dout), lambda i: (0, 0)),
            pl.BlockSpec((1, dout), lambda i: (0, 0)),
        ],
        out_specs=[
            pl.BlockSpec((blk, dout), lambda i: (i, 0)),
            pl.BlockSpec((blk, dout), lambda i: (i, 0)),
        ],
        out_shape=[jax.ShapeDtypeStruct((n, dout), jnp.float32)] * 2,
    )(h, W_self, W_neigh, b.reshape(1, dout))


def _combine_transform_body(hs_ref, agg_ref, deg_ref, ws_ref, wn_ref, b_ref,
                            hs2_ref, ht2_ref):
    agg = agg_ref[0] + agg_ref[1]
    deg = deg_ref[0] + deg_ref[1]
    inv = 1.0 / jnp.maximum(deg, 1.0)
    h = jnp.maximum(hs_ref[...] + agg * inv, 0.0)
    hs2_ref[...] = jnp.dot(h, ws_ref[...], preferred_element_type=jnp.float32) + b_ref[...]
    ht2_ref[...] = jnp.dot(h, wn_ref[...], preferred_element_type=jnp.float32)


def _combine_transform(hs, aggp, degp, W_self, W_neigh, b, blk=512):
    n, d = hs.shape
    dout = W_self.shape[1]
    return pl.pallas_call(
        _combine_transform_body,
        grid=(pl.cdiv(n, blk),),
        in_specs=[
            pl.BlockSpec((blk, d), lambda i: (i, 0)),
            pl.BlockSpec((NC, blk, d), lambda i: (0, i, 0)),
            pl.BlockSpec((NC, blk, d), lambda i: (0, i, 0)),
            pl.BlockSpec((d, dout), lambda i: (0, 0)),
            pl.BlockSpec((d, dout), lambda i: (0, 0)),
            pl.BlockSpec((1, dout), lambda i: (0, 0)),
        ],
        out_specs=[
            pl.BlockSpec((blk, dout), lambda i: (i, 0)),
            pl.BlockSpec((blk, dout), lambda i: (i, 0)),
        ],
        out_shape=[jax.ShapeDtypeStruct((n, dout), jnp.float32)] * 2,
    )(hs, aggp, degp, W_self, W_neigh, b.reshape(1, dout))


def _combine_keep_body(hs_ref, agg_ref, deg_ref, ws_ref, b_ref,
                       hs2_ref, h_ref):
    agg = agg_ref[0] + agg_ref[1]
    deg = deg_ref[0] + deg_ref[1]
    inv = 1.0 / jnp.maximum(deg, 1.0)
    h = jnp.maximum(hs_ref[...] + agg * inv, 0.0)
    hs2_ref[...] = jnp.dot(h, ws_ref[...], preferred_element_type=jnp.float32) + b_ref[...]
    h_ref[...] = h


def _combine_keep(hs, aggp, degp, W_self, b, blk=512):
    n, d = hs.shape
    dout = W_self.shape[1]
    return pl.pallas_call(
        _combine_keep_body,
        grid=(pl.cdiv(n, blk),),
        in_specs=[
            pl.BlockSpec((blk, d), lambda i: (i, 0)),
            pl.BlockSpec((NC, blk, d), lambda i: (0, i, 0)),
            pl.BlockSpec((NC, blk, d), lambda i: (0, i, 0)),
            pl.BlockSpec((d, dout), lambda i: (0, 0)),
            pl.BlockSpec((1, dout), lambda i: (0, 0)),
        ],
        out_specs=[
            pl.BlockSpec((blk, dout), lambda i: (i, 0)),
            pl.BlockSpec((blk, d), lambda i: (i, 0)),
        ],
        out_shape=[jax.ShapeDtypeStruct((n, dout), jnp.float32),
                   jax.ShapeDtypeStruct((n, d), jnp.float32)],
    )(hs, aggp, degp, W_self, b.reshape(1, dout))


def _final_body(hs_ref, agg_ref, deg_ref, wn_ref, out_ref):
    agg = agg_ref[0] + agg_ref[1]
    deg = deg_ref[0] + deg_ref[1]
    inv = 1.0 / jnp.maximum(deg, 1.0)
    out_ref[...] = hs_ref[...] + jnp.dot(
        agg * inv, wn_ref[...], preferred_element_type=jnp.float32)


def _final(hs, aggp, degp, W_neigh, blk=512):
    n, d = hs.shape
    dagg = aggp.shape[2]
    return pl.pallas_call(
        _final_body,
        grid=(pl.cdiv(n, blk),),
        in_specs=[
            pl.BlockSpec((blk, d), lambda i: (i, 0)),
            pl.BlockSpec((NC, blk, dagg), lambda i: (0, i, 0)),
            pl.BlockSpec((NC, blk, dagg), lambda i: (0, i, 0)),
            pl.BlockSpec((dagg, d), lambda i: (0, 0)),
        ],
        out_specs=pl.BlockSpec((blk, d), lambda i: (i, 0)),
        out_shape=jax.ShapeDtypeStruct((n, d), jnp.float32),
    )(hs, aggp, degp, W_neigh)


def _pick_layout(epw, d, npad, with_gather):
    for ch in (128, 120, 112, 104, 96, 88, 80, 72, 64, 56, 48, 40, 32, 24,
               16, 8):
        if epw % ch:
            continue
        steps = epw // ch
        for nb in (5, 4, 6, 3, 2):
            if steps % nb:
                continue
            words = npad * d + NS * (nb * ch * (d + 1)
                                     + (epw if with_gather else 0))
            if words <= 2_000_000:
                return ch, nb
    raise ValueError("no feasible SC layout")


@functools.lru_cache(maxsize=None)
def _make_agg(n, e, d, ones_table=False):
    nw = NC * NS
    epw = e // nw
    rpt = ((n + NS - 1) // NS + CPY - 1) // CPY * CPY
    npad = rpt * NS
    ch, nbuf = _pick_layout(epw, d, npad, not ones_table)
    steps = epw // ch
    cpy = CPY if (ch >= CPY and rpt % CPY == 0) else ch
    ncopy = rpt // cpy

    mesh = plsc.VectorSubcoreMesh(core_axis_name="c", subcore_axis_name="s",
                                  num_cores=NC, num_subcores=NS)

    scratch = [
        pltpu.VMEM_SHARED((npad, d), jnp.float32),
        [pltpu.VMEM((ch,), jnp.int32) for _ in range(nbuf)],
        [pltpu.VMEM((ch, d), jnp.float32) for _ in range(nbuf)],
        [pltpu.SemaphoreType.DMA for _ in range(nbuf)],
    ]
    if not ones_table:
        scratch += [
            pltpu.VMEM((epw,), jnp.int32),
            [pltpu.SemaphoreType.DMA for _ in range(nbuf)],
        ]

    def body(table, srcv, dstv, zeros_nd, agg_out,
             acc, idx_d, rows, isem, srcall, gsem):
        c = lax.axis_index("c")
        s = lax.axis_index("s")
        wid = c * NS + s
        r0 = s * rpt
        base = wid * epw
        pltpu.sync_copy(zeros_nd.at[pl.ds(0, cpy)], rows[0].at[pl.ds(0, cpy)])
        for k in range(ncopy):
            pltpu.sync_copy(rows[0].at[pl.ds(0, cpy)],
                            acc.at[pl.ds(r0 + k * cpy, cpy)])
        if ones_table:
            for b in range(nbuf):
                pltpu.sync_copy(table.at[pl.ds(0, ch)], rows[b])
        else:
            pltpu.sync_copy(srcv.at[pl.ds(base, epw)], srcall)
        plsc.subcore_barrier()

        def super_step(g, carry):
            j0 = g * nbuf
            descs = []
            for b in range(nbuf):
                off = base + (j0 + b) * ch
                di = pltpu.async_copy(dstv.at[pl.ds(off, ch)], idx_d[b],
                                      isem[b])
                if ones_table:
                    descs.append((di, None))
                else:
                    gi = pltpu.async_copy(
                        table.at[srcall.at[pl.ds((j0 + b) * ch, ch)]],
                        rows[b], gsem[b])
                    descs.append((di, gi))
            for b, (di, gi) in enumerate(descs):
                di.wait()
                if gi is not None:
                    gi.wait()
                pltpu.sync_copy(rows[b], acc.at[idx_d[b]], add=True)
            return carry

        lax.fori_loop(0, steps // nbuf, super_step, 0)
        plsc.subcore_barrier()
        for k in range(ncopy):
            pltpu.sync_copy(acc.at[pl.ds(r0 + k * cpy, cpy)],
                            rows[0].at[pl.ds(0, cpy)])
            pltpu.sync_copy(rows[0].at[pl.ds(0, cpy)],
                            agg_out.at[c].at[pl.ds(r0 + k * cpy, cpy)])

    out_type = [jax.ShapeDtypeStruct((NC, npad, d), jnp.float32)]
    if ones_table:
        def fn(table, dstv, zeros_nd, agg_out, acc, idx_d, rows, isem):
            body(table, None, dstv, zeros_nd, agg_out,
                 acc, idx_d, rows, isem, None, None)

        kern = functools.partial(pl.kernel, mesh=mesh, out_type=out_type,
                                 scratch_types=scratch)(fn)

        def run(table, src, dst):
            del src
            zeros_nd = jnp.zeros((npad, d), jnp.float32)
            return kern(table, dst, zeros_nd)[0]
    else:
        def fn(table, srcv, dstv, zeros_nd, agg_out, acc, idx_d, rows, isem,
               srcall, gsem):
            body(table, srcv, dstv, zeros_nd, agg_out,
                 acc, idx_d, rows, isem, srcall, gsem)

        kern = functools.partial(pl.kernel, mesh=mesh, out_type=out_type,
                                 scratch_types=scratch)(fn)

        def run(table, src, dst):
            zeros_nd = jnp.zeros((npad, d), jnp.float32)
            return kern(table, src, dst, zeros_nd)[0]

    return run


def kernel(x, edge_index, W_self0, W_neigh0, b0, W_self1, W_neigh1, b1,
           W_self2, W_neigh2, b2):
    src = edge_index[0]
    dst = edge_index[1]
    n, _ = x.shape
    e = src.shape[0]
    d_hid = W_self0.shape[1]

    agg_hid = _make_agg(n, e, d_hid)

    degp = _make_agg(n, e, d_hid, ones_table=True)(
        jnp.ones((n, d_hid), jnp.float32), src, dst)

    hs0, ht0 = _transform(x, W_self0, W_neigh0, b0)
    aggp0 = agg_hid(ht0, src, dst)
    hs1, ht1 = _combine_transform(hs0, aggp0, degp, W_self1, W_neigh1, b1)
    aggp1 = agg_hid(ht1, src, dst)
    hs2, h2 = _combine_keep(hs1, aggp1, degp, W_self2, b2)
    aggp2 = agg_hid(h2, src, dst)
    return _final(hs2, aggp2, degp, W_neigh2)


# speedup vs baseline: 8.8902x; 1.0588x over previous
import functools

import jax
import jax.numpy as jnp
from jax import lax
from jax.experimental import pallas as pl
from jax.experimental.pallas import tpu as pltpu
from jax.experimental.pallas import tpu_sc as plsc

NC = 2
NS = 16
CPY = 64


def _transform_body(h_ref, ws_ref, wn_ref, b_ref, hs_ref, ht_ref):
    h = h_ref[...]
    hs_ref[...] = jnp.dot(h, ws_ref[...], preferred_element_type=jnp.float32) + b_ref[...]
    ht_ref[...] = jnp.dot(h, wn_ref[...], preferred_element_type=jnp.float32)


def _transform(h, W_self, W_neigh, b, blk=512):
    n, d = h.shape
    dout = W_self.shape[1]
    return pl.pallas_call(
        _transform_body,
        grid=(pl.cdiv(n, blk),),
        in_specs=[
            pl.BlockSpec((blk, d), lambda i: (i, 0)),
            pl.BlockSpec((d, dout), lambda i: (0, 0)),
            pl.BlockSpec((d, dout), lambda i: (0, 0)),
            pl.BlockSpec((1, dout), lambda i: (0, 0)),
        ],
        out_specs=[
            pl.BlockSpec((blk, dout), lambda i: (i, 0)),
            pl.BlockSpec((blk, dout), lambda i: (i, 0)),
        ],
        out_shape=[jax.ShapeDtypeStruct((n, dout), jnp.float32)] * 2,
    )(h, W_self, W_neigh, b.reshape(1, dout))


def _combine_transform_body(hs_ref, agg_ref, deg_ref, ws_ref, wn_ref, b_ref,
                            hs2_ref, ht2_ref):
    agg = agg_ref[0] + agg_ref[1]
    deg = deg_ref[0] + deg_ref[1]
    inv = 1.0 / jnp.maximum(deg, 1.0)
    h = jnp.maximum(hs_ref[...] + agg * inv, 0.0)
    hs2_ref[...] = jnp.dot(h, ws_ref[...], preferred_element_type=jnp.float32) + b_ref[...]
    ht2_ref[...] = jnp.dot(h, wn_ref[...], preferred_element_type=jnp.float32)


def _combine_transform(hs, aggp, degp, W_self, W_neigh, b, blk=512):
    n, d = hs.shape
    dout = W_self.shape[1]
    return pl.pallas_call(
        _combine_transform_body,
        grid=(pl.cdiv(n, blk),),
        in_specs=[
            pl.BlockSpec((blk, d), lambda i: (i, 0)),
            pl.BlockSpec((NC, blk, d), lambda i: (0, i, 0)),
            pl.BlockSpec((NC, blk, d), lambda i: (0, i, 0)),
            pl.BlockSpec((d, dout), lambda i: (0, 0)),
            pl.BlockSpec((d, dout), lambda i: (0, 0)),
            pl.BlockSpec((1, dout), lambda i: (0, 0)),
        ],
        out_specs=[
            pl.BlockSpec((blk, dout), lambda i: (i, 0)),
            pl.BlockSpec((blk, dout), lambda i: (i, 0)),
        ],
        out_shape=[jax.ShapeDtypeStruct((n, dout), jnp.float32)] * 2,
    )(hs, aggp, degp, W_self, W_neigh, b.reshape(1, dout))


def _combine_keep_body(hs_ref, agg_ref, deg_ref, ws_ref, b_ref,
                       hs2_ref, h_ref):
    agg = agg_ref[0] + agg_ref[1]
    deg = deg_ref[0] + deg_ref[1]
    inv = 1.0 / jnp.maximum(deg, 1.0)
    h = jnp.maximum(hs_ref[...] + agg * inv, 0.0)
    hs2_ref[...] = jnp.dot(h, ws_ref[...], preferred_element_type=jnp.float32) + b_ref[...]
    h_ref[...] = h


def _combine_keep(hs, aggp, degp, W_self, b, blk=512):
    n, d = hs.shape
    dout = W_self.shape[1]
    return pl.pallas_call(
        _combine_keep_body,
        grid=(pl.cdiv(n, blk),),
        in_specs=[
            pl.BlockSpec((blk, d), lambda i: (i, 0)),
            pl.BlockSpec((NC, blk, d), lambda i: (0, i, 0)),
            pl.BlockSpec((NC, blk, d), lambda i: (0, i, 0)),
            pl.BlockSpec((d, dout), lambda i: (0, 0)),
            pl.BlockSpec((1, dout), lambda i: (0, 0)),
        ],
        out_specs=[
            pl.BlockSpec((blk, dout), lambda i: (i, 0)),
            pl.BlockSpec((blk, d), lambda i: (i, 0)),
        ],
        out_shape=[jax.ShapeDtypeStruct((n, dout), jnp.float32),
                   jax.ShapeDtypeStruct((n, d), jnp.float32)],
    )(hs, aggp, degp, W_self, b.reshape(1, dout))


def _final_body(hs_ref, agg_ref, deg_ref, wn_ref, out_ref):
    agg = agg_ref[0] + agg_ref[1]
    deg = deg_ref[0] + deg_ref[1]
    inv = 1.0 / jnp.maximum(deg, 1.0)
    out_ref[...] = hs_ref[...] + jnp.dot(
        agg * inv, wn_ref[...], preferred_element_type=jnp.float32)


def _final(hs, aggp, degp, W_neigh, blk=512):
    n, d = hs.shape
    dagg = aggp.shape[2]
    return pl.pallas_call(
        _final_body,
        grid=(pl.cdiv(n, blk),),
        in_specs=[
            pl.BlockSpec((blk, d), lambda i: (i, 0)),
            pl.BlockSpec((NC, blk, dagg), lambda i: (0, i, 0)),
            pl.BlockSpec((NC, blk, dagg), lambda i: (0, i, 0)),
            pl.BlockSpec((dagg, d), lambda i: (0, 0)),
        ],
        out_specs=pl.BlockSpec((blk, d), lambda i: (i, 0)),
        out_shape=jax.ShapeDtypeStruct((n, d), jnp.float32),
    )(hs, aggp, degp, W_neigh)


def _pick_layout(epw, d, npad, with_gather):
    for ch in (128, 120, 112, 104, 96, 88, 80, 72, 64, 56, 48, 40, 32, 24,
               16, 8):
        if epw % ch:
            continue
        steps = epw // ch
        for nb in (5, 4, 6, 3, 2):
            if steps % nb:
                continue
            words = npad * d + NS * (nb * ch * (d + 1)
                                     + (epw if with_gather else 0))
            if words <= 2_000_000:
                return ch, nb
    raise ValueError("no feasible SC layout")


@functools.lru_cache(maxsize=None)
def _make_agg(n, e, d, ones_table=False):
    nw = NC * NS
    epw = e // nw
    rpt = ((n + NS - 1) // NS + CPY - 1) // CPY * CPY
    npad = rpt * NS
    ch, nbuf = _pick_layout(epw, d, npad, not ones_table)
    steps = epw // ch
    cpy = CPY if (ch >= CPY and rpt % CPY == 0) else ch
    ncopy = rpt // cpy

    mesh = plsc.VectorSubcoreMesh(core_axis_name="c", subcore_axis_name="s",
                                  num_cores=NC, num_subcores=NS)

    scratch = [
        pltpu.VMEM_SHARED((npad, d), jnp.float32),
        [pltpu.VMEM((ch,), jnp.int32) for _ in range(nbuf)],
        [pltpu.VMEM((ch, d), jnp.float32) for _ in range(nbuf)],
        [pltpu.SemaphoreType.DMA for _ in range(nbuf)],
        [pltpu.SemaphoreType.DMA for _ in range(nbuf)],
    ]
    if not ones_table:
        scratch += [
            pltpu.VMEM((epw,), jnp.int32),
            [pltpu.SemaphoreType.DMA for _ in range(nbuf)],
        ]

    def body(table, srcv, dstv, zeros_nd, agg_out,
             acc, idx_d, rows, isem, ssem, srcall, gsem):
        c = lax.axis_index("c")
        s = lax.axis_index("s")
        wid = c * NS + s
        r0 = s * rpt
        base = wid * epw
        pltpu.sync_copy(zeros_nd.at[pl.ds(0, cpy)], rows[0].at[pl.ds(0, cpy)])
        for k in range(ncopy):
            pltpu.sync_copy(rows[0].at[pl.ds(0, cpy)],
                            acc.at[pl.ds(r0 + k * cpy, cpy)])
        if ones_table:
            for b in range(nbuf):
                pltpu.sync_copy(table.at[pl.ds(0, ch)], rows[b])
        else:
            pltpu.sync_copy(srcv.at[pl.ds(base, epw)], srcall)
        plsc.subcore_barrier()

        def super_step(g, carry):
            j0 = g * nbuf
            descs = []
            for b in range(nbuf):
                off = base + (j0 + b) * ch
                di = pltpu.async_copy(dstv.at[pl.ds(off, ch)], idx_d[b],
                                      isem[b])
                if ones_table:
                    descs.append((di, None))
                else:
                    gi = pltpu.async_copy(
                        table.at[srcall.at[pl.ds((j0 + b) * ch, ch)]],
                        rows[b], gsem[b])
                    descs.append((di, gi))
            sdescs = []
            for b, (di, gi) in enumerate(descs):
                di.wait()
                if gi is not None:
                    gi.wait()
                sdescs.append(pltpu.async_copy(rows[b], acc.at[idx_d[b]],
                                               ssem[b], add=True))
            for sd in sdescs:
                sd.wait()
            return carry

        lax.fori_loop(0, steps // nbuf, super_step, 0)
        plsc.subcore_barrier()
        for k in range(ncopy):
            pltpu.sync_copy(acc.at[pl.ds(r0 + k * cpy, cpy)],
                            rows[0].at[pl.ds(0, cpy)])
            pltpu.sync_copy(rows[0].at[pl.ds(0, cpy)],
                            agg_out.at[c].at[pl.ds(r0 + k * cpy, cpy)])

    out_type = [jax.ShapeDtypeStruct((NC, npad, d), jnp.float32)]
    if ones_table:
        def fn(table, dstv, zeros_nd, agg_out, acc, idx_d, rows, isem, ssem):
            body(table, None, dstv, zeros_nd, agg_out,
                 acc, idx_d, rows, isem, ssem, None, None)

        kern = functools.partial(pl.kernel, mesh=mesh, out_type=out_type,
                                 scratch_types=scratch)(fn)

        def run(table, src, dst):
            del src
            zeros_nd = jnp.zeros((npad, d), jnp.float32)
            return kern(table, dst, zeros_nd)[0]
    else:
        def fn(table, srcv, dstv, zeros_nd, agg_out, acc, idx_d, rows, isem,
               ssem, srcall, gsem):
            body(table, srcv, dstv, zeros_nd, agg_out,
                 acc, idx_d, rows, isem, ssem, srcall, gsem)

        kern = functools.partial(pl.kernel, mesh=mesh, out_type=out_type,
                                 scratch_types=scratch)(fn)

        def run(table, src, dst):
            zeros_nd = jnp.zeros((npad, d), jnp.float32)
            return kern(table, src, dst, zeros_nd)[0]

    return run


def kernel(x, edge_index, W_self0, W_neigh0, b0, W_self1, W_neigh1, b1,
           W_self2, W_neigh2, b2):
    src = edge_index[0]
    dst = edge_index[1]
    n, _ = x.shape
    e = src.shape[0]
    d_hid = W_self0.shape[1]

    agg_hid = _make_agg(n, e, d_hid)

    degp = _make_agg(n, e, d_hid, ones_table=True)(
        jnp.ones((n, d_hid), jnp.float32), src, dst)

    hs0, ht0 = _transform(x, W_self0, W_neigh0, b0)
    aggp0 = agg_hid(ht0, src, dst)
    hs1, ht1 = _combine_transform(hs0, aggp0, degp, W_self1, W_neigh1, b1)
    aggp1 = agg_hid(ht1, src, dst)
    hs2, h2 = _combine_keep(hs1, aggp1, degp, W_self2, b2)
    aggp2 = agg_hid(h2, src, dst)
    return _final(hs2, aggp2, degp, W_neigh2)


# speedup vs baseline: 8.9787x; 1.0100x over previous
import functools

import jax
import jax.numpy as jnp
from jax import lax
from jax.experimental import pallas as pl
from jax.experimental.pallas import tpu as pltpu
from jax.experimental.pallas import tpu_sc as plsc

NC = 2
NS = 16
CPY = 64


def _transform_body(h_ref, ws_ref, wn_ref, b_ref, hs_ref, ht_ref):
    h = h_ref[...]
    hs_ref[...] = jnp.dot(h, ws_ref[...], preferred_element_type=jnp.float32) + b_ref[...]
    ht_ref[...] = jnp.dot(h, wn_ref[...], preferred_element_type=jnp.float32)


def _transform(h, W_self, W_neigh, b, blk=512):
    n, d = h.shape
    dout = W_self.shape[1]
    return pl.pallas_call(
        _transform_body,
        grid=(pl.cdiv(n, blk),),
        in_specs=[
            pl.BlockSpec((blk, d), lambda i: (i, 0)),
            pl.BlockSpec((d, dout), lambda i: (0, 0)),
            pl.BlockSpec((d, dout), lambda i: (0, 0)),
            pl.BlockSpec((1, dout), lambda i: (0, 0)),
        ],
        out_specs=[
            pl.BlockSpec((blk, dout), lambda i: (i, 0)),
            pl.BlockSpec((blk, dout), lambda i: (i, 0)),
        ],
        out_shape=[jax.ShapeDtypeStruct((n, dout), jnp.float32)] * 2,
    )(h, W_self, W_neigh, b.reshape(1, dout))


def _combine_transform_body(hs_ref, agg_ref, deg_ref, ws_ref, wn_ref, b_ref,
                            hs2_ref, ht2_ref):
    agg = agg_ref[0] + agg_ref[1]
    deg = deg_ref[0] + deg_ref[1]
    inv = 1.0 / jnp.maximum(deg, 1.0)
    h = jnp.maximum(hs_ref[...] + agg * inv, 0.0)
    hs2_ref[...] = jnp.dot(h, ws_ref[...], preferred_element_type=jnp.float32) + b_ref[...]
    ht2_ref[...] = jnp.dot(h, wn_ref[...], preferred_element_type=jnp.float32)


def _combine_transform(hs, aggp, degp, W_self, W_neigh, b, blk=512):
    n, d = hs.shape
    dout = W_self.shape[1]
    return pl.pallas_call(
        _combine_transform_body,
        grid=(pl.cdiv(n, blk),),
        in_specs=[
            pl.BlockSpec((blk, d), lambda i: (i, 0)),
            pl.BlockSpec((NC, blk, d), lambda i: (0, i, 0)),
            pl.BlockSpec((NC, blk, d), lambda i: (0, i, 0)),
            pl.BlockSpec((d, dout), lambda i: (0, 0)),
            pl.BlockSpec((d, dout), lambda i: (0, 0)),
            pl.BlockSpec((1, dout), lambda i: (0, 0)),
        ],
        out_specs=[
            pl.BlockSpec((blk, dout), lambda i: (i, 0)),
            pl.BlockSpec((blk, dout), lambda i: (i, 0)),
        ],
        out_shape=[jax.ShapeDtypeStruct((n, dout), jnp.float32)] * 2,
    )(hs, aggp, degp, W_self, W_neigh, b.reshape(1, dout))


def _combine_keep_body(hs_ref, agg_ref, deg_ref, ws_ref, b_ref,
                       hs2_ref, h_ref):
    agg = agg_ref[0] + agg_ref[1]
    deg = deg_ref[0] + deg_ref[1]
    inv = 1.0 / jnp.maximum(deg, 1.0)
    h = jnp.maximum(hs_ref[...] + agg * inv, 0.0)
    hs2_ref[...] = jnp.dot(h, ws_ref[...], preferred_element_type=jnp.float32) + b_ref[...]
    h_ref[...] = h


def _combine_keep(hs, aggp, degp, W_self, b, blk=512):
    n, d = hs.shape
    dout = W_self.shape[1]
    return pl.pallas_call(
        _combine_keep_body,
        grid=(pl.cdiv(n, blk),),
        in_specs=[
            pl.BlockSpec((blk, d), lambda i: (i, 0)),
            pl.BlockSpec((NC, blk, d), lambda i: (0, i, 0)),
            pl.BlockSpec((NC, blk, d), lambda i: (0, i, 0)),
            pl.BlockSpec((d, dout), lambda i: (0, 0)),
            pl.BlockSpec((1, dout), lambda i: (0, 0)),
        ],
        out_specs=[
            pl.BlockSpec((blk, dout), lambda i: (i, 0)),
            pl.BlockSpec((blk, d), lambda i: (i, 0)),
        ],
        out_shape=[jax.ShapeDtypeStruct((n, dout), jnp.float32),
                   jax.ShapeDtypeStruct((n, d), jnp.float32)],
    )(hs, aggp, degp, W_self, b.reshape(1, dout))


def _final_body(hs_ref, agg_ref, deg_ref, wn_ref, out_ref):
    agg = agg_ref[0] + agg_ref[1]
    deg = deg_ref[0] + deg_ref[1]
    inv = 1.0 / jnp.maximum(deg, 1.0)
    out_ref[...] = hs_ref[...] + jnp.dot(
        agg * inv, wn_ref[...], preferred_element_type=jnp.float32)


def _final(hs, aggp, degp, W_neigh, blk=512):
    n, d = hs.shape
    dagg = aggp.shape[2]
    return pl.pallas_call(
        _final_body,
        grid=(pl.cdiv(n, blk),),
        in_specs=[
            pl.BlockSpec((blk, d), lambda i: (i, 0)),
            pl.BlockSpec((NC, blk, dagg), lambda i: (0, i, 0)),
            pl.BlockSpec((NC, blk, dagg), lambda i: (0, i, 0)),
            pl.BlockSpec((dagg, d), lambda i: (0, 0)),
        ],
        out_specs=pl.BlockSpec((blk, d), lambda i: (i, 0)),
        out_shape=jax.ShapeDtypeStruct((n, d), jnp.float32),
    )(hs, aggp, degp, W_neigh)


def _pick_layout(epw, d, npad, with_gather):
    for ch in (128, 120, 112, 104, 96, 88, 80, 72, 64, 56, 48, 40, 32, 24,
               16, 8):
        if epw % ch:
            continue
        steps = epw // ch
        for nb in (5, 4, 6, 3, 2):
            if steps % nb:
                continue
            words = npad * d + NS * (nb * ch * (d + 1)
                                     + (epw if with_gather else 0))
            if words <= 2_000_000:
                return ch, nb
    raise ValueError("no feasible SC layout")


@functools.lru_cache(maxsize=None)
def _make_agg(n, e, d, ones_table=False):
    nw = NC * NS
    epw = e // nw
    rpt = ((n + NS - 1) // NS + CPY - 1) // CPY * CPY
    npad = rpt * NS
    ch, nbuf = _pick_layout(epw, d, npad, not ones_table)
    steps = epw // ch
    cpy = CPY if (ch >= CPY and rpt % CPY == 0) else ch
    ncopy = rpt // cpy

    mesh = plsc.VectorSubcoreMesh(core_axis_name="c", subcore_axis_name="s",
                                  num_cores=NC, num_subcores=NS)

    scratch = [
        pltpu.VMEM_SHARED((npad, d), jnp.float32),
        [pltpu.VMEM((ch,), jnp.int32) for _ in range(nbuf)],
        [pltpu.VMEM((ch, d), jnp.float32) for _ in range(nbuf)],
        [pltpu.SemaphoreType.DMA for _ in range(nbuf)],
        [pltpu.SemaphoreType.DMA for _ in range(nbuf)],
    ]
    if not ones_table:
        scratch += [
            pltpu.VMEM((epw,), jnp.int32),
            [pltpu.SemaphoreType.DMA for _ in range(nbuf)],
        ]

    def body(table, srcv, dstv, zeros_nd, agg_out,
             acc, idx_d, rows, isem, ssem, srcall, gsem):
        c = lax.axis_index("c")
        s = lax.axis_index("s")
        wid = c * NS + s
        r0 = s * rpt
        base = wid * epw
        pltpu.sync_copy(zeros_nd.at[pl.ds(r0, rpt)], acc.at[pl.ds(r0, rpt)])
        if ones_table:
            for b in range(nbuf):
                pltpu.sync_copy(table.at[pl.ds(0, ch)], rows[b])
        else:
            pltpu.sync_copy(srcv.at[pl.ds(base, epw)], srcall)
        plsc.subcore_barrier()

        def super_step(g, carry):
            j0 = g * nbuf
            descs = []
            for b in range(nbuf):
                off = base + (j0 + b) * ch
                di = pltpu.async_copy(dstv.at[pl.ds(off, ch)], idx_d[b],
                                      isem[b])
                if ones_table:
                    descs.append((di, None))
                else:
                    gi = pltpu.async_copy(
                        table.at[srcall.at[pl.ds((j0 + b) * ch, ch)]],
                        rows[b], gsem[b])
                    descs.append((di, gi))
            sdescs = []
            for b, (di, gi) in enumerate(descs):
                di.wait()
                if gi is not None:
                    gi.wait()
                sdescs.append(pltpu.async_copy(rows[b], acc.at[idx_d[b]],
                                               ssem[b], add=True))
            for sd in sdescs:
                sd.wait()
            return carry

        lax.fori_loop(0, steps // nbuf, super_step, 0)
        plsc.subcore_barrier()
        pltpu.sync_copy(acc.at[pl.ds(r0, rpt)],
                        agg_out.at[c].at[pl.ds(r0, rpt)])

    out_type = [jax.ShapeDtypeStruct((NC, npad, d), jnp.float32)]
    if ones_table:
        def fn(table, dstv, zeros_nd, agg_out, acc, idx_d, rows, isem, ssem):
            body(table, None, dstv, zeros_nd, agg_out,
                 acc, idx_d, rows, isem, ssem, None, None)

        kern = functools.partial(pl.kernel, mesh=mesh, out_type=out_type,
                                 scratch_types=scratch)(fn)

        def run(table, src, dst):
            del src
            zeros_nd = jnp.zeros((npad, d), jnp.float32)
            return kern(table, dst, zeros_nd)[0]
    else:
        def fn(table, srcv, dstv, zeros_nd, agg_out, acc, idx_d, rows, isem,
               ssem, srcall, gsem):
            body(table, srcv, dstv, zeros_nd, agg_out,
                 acc, idx_d, rows, isem, ssem, srcall, gsem)

        kern = functools.partial(pl.kernel, mesh=mesh, out_type=out_type,
                                 scratch_types=scratch)(fn)

        def run(table, src, dst):
            zeros_nd = jnp.zeros((npad, d), jnp.float32)
            return kern(table, src, dst, zeros_nd)[0]

    return run


def kernel(x, edge_index, W_self0, W_neigh0, b0, W_self1, W_neigh1, b1,
           W_self2, W_neigh2, b2):
    src = edge_index[0]
    dst = edge_index[1]
    n, _ = x.shape
    e = src.shape[0]
    d_hid = W_self0.shape[1]

    agg_hid = _make_agg(n, e, d_hid)

    degp = _make_agg(n, e, d_hid, ones_table=True)(
        jnp.ones((n, d_hid), jnp.float32), src, dst)

    hs0, ht0 = _transform(x, W_self0, W_neigh0, b0)
    aggp0 = agg_hid(ht0, src, dst)
    hs1, ht1 = _combine_transform(hs0, aggp0, degp, W_self1, W_neigh1, b1)
    aggp1 = agg_hid(ht1, src, dst)
    hs2, h2 = _combine_keep(hs1, aggp1, degp, W_self2, b2)
    aggp2 = agg_hid(h2, src, dst)
    return _final(hs2, aggp2, degp, W_neigh2)


# speedup vs baseline: 9.7469x; 1.0856x over previous
import functools

import jax
import jax.numpy as jnp
from jax import lax
from jax.experimental import pallas as pl
from jax.experimental.pallas import tpu as pltpu
from jax.experimental.pallas import tpu_sc as plsc

NC = 2
NS = 16
CPY = 64


def _transform_body(h_ref, ws_ref, wn_ref, b_ref, hs_ref, ht_ref):
    h = h_ref[...]
    hs_ref[...] = jnp.dot(h, ws_ref[...], preferred_element_type=jnp.float32) + b_ref[...]
    ht_ref[...] = jnp.dot(h, wn_ref[...], preferred_element_type=jnp.float32)


def _transform(h, W_self, W_neigh, b, blk=512):
    n, d = h.shape
    dout = W_self.shape[1]
    return pl.pallas_call(
        _transform_body,
        grid=(pl.cdiv(n, blk),),
        in_specs=[
            pl.BlockSpec((blk, d), lambda i: (i, 0)),
            pl.BlockSpec((d, dout), lambda i: (0, 0)),
            pl.BlockSpec((d, dout), lambda i: (0, 0)),
            pl.BlockSpec((1, dout), lambda i: (0, 0)),
        ],
        out_specs=[
            pl.BlockSpec((blk, dout), lambda i: (i, 0)),
            pl.BlockSpec((blk, dout), lambda i: (i, 0)),
        ],
        out_shape=[jax.ShapeDtypeStruct((n, dout), jnp.float32)] * 2,
    )(h, W_self, W_neigh, b.reshape(1, dout))


def _combine_transform_body(hs_ref, agg_ref, deg_ref, ws_ref, wn_ref, b_ref,
                            hs2_ref, ht2_ref):
    agg = agg_ref[0] + agg_ref[1]
    deg = deg_ref[0] + deg_ref[1]
    inv = 1.0 / jnp.maximum(deg, 1.0)
    h = jnp.maximum(hs_ref[...] + agg * inv, 0.0)
    hs2_ref[...] = jnp.dot(h, ws_ref[...], preferred_element_type=jnp.float32) + b_ref[...]
    ht2_ref[...] = jnp.dot(h, wn_ref[...], preferred_element_type=jnp.float32)


def _combine_transform(hs, aggp, degp, W_self, W_neigh, b, blk=512):
    n, d = hs.shape
    dout = W_self.shape[1]
    return pl.pallas_call(
        _combine_transform_body,
        grid=(pl.cdiv(n, blk),),
        in_specs=[
            pl.BlockSpec((blk, d), lambda i: (i, 0)),
            pl.BlockSpec((NC, blk, d), lambda i: (0, i, 0)),
            pl.BlockSpec((NC, blk, d), lambda i: (0, i, 0)),
            pl.BlockSpec((d, dout), lambda i: (0, 0)),
            pl.BlockSpec((d, dout), lambda i: (0, 0)),
            pl.BlockSpec((1, dout), lambda i: (0, 0)),
        ],
        out_specs=[
            pl.BlockSpec((blk, dout), lambda i: (i, 0)),
            pl.BlockSpec((blk, dout), lambda i: (i, 0)),
        ],
        out_shape=[jax.ShapeDtypeStruct((n, dout), jnp.float32)] * 2,
    )(hs, aggp, degp, W_self, W_neigh, b.reshape(1, dout))


def _combine_keep_body(hs_ref, agg_ref, deg_ref, ws_ref, b_ref,
                       hs2_ref, h_ref):
    agg = agg_ref[0] + agg_ref[1]
    deg = deg_ref[0] + deg_ref[1]
    inv = 1.0 / jnp.maximum(deg, 1.0)
    h = jnp.maximum(hs_ref[...] + agg * inv, 0.0)
    hs2_ref[...] = jnp.dot(h, ws_ref[...], preferred_element_type=jnp.float32) + b_ref[...]
    h_ref[...] = h


def _combine_keep(hs, aggp, degp, W_self, b, blk=512):
    n, d = hs.shape
    dout = W_self.shape[1]
    return pl.pallas_call(
        _combine_keep_body,
        grid=(pl.cdiv(n, blk),),
        in_specs=[
            pl.BlockSpec((blk, d), lambda i: (i, 0)),
            pl.BlockSpec((NC, blk, d), lambda i: (0, i, 0)),
            pl.BlockSpec((NC, blk, d), lambda i: (0, i, 0)),
            pl.BlockSpec((d, dout), lambda i: (0, 0)),
            pl.BlockSpec((1, dout), lambda i: (0, 0)),
        ],
        out_specs=[
            pl.BlockSpec((blk, dout), lambda i: (i, 0)),
            pl.BlockSpec((blk, d), lambda i: (i, 0)),
        ],
        out_shape=[jax.ShapeDtypeStruct((n, dout), jnp.float32),
                   jax.ShapeDtypeStruct((n, d), jnp.float32)],
    )(hs, aggp, degp, W_self, b.reshape(1, dout))


def _final_body(hs_ref, agg_ref, deg_ref, out_ref):
    agg = agg_ref[0] + agg_ref[1]
    deg = deg_ref[0][:, :agg.shape[1]] + deg_ref[1][:, :agg.shape[1]]
    inv = 1.0 / jnp.maximum(deg, 1.0)
    out_ref[...] = hs_ref[...] + agg * inv


def _final(hs, aggp, degp, blk=512):
    n, d = hs.shape
    ddeg = degp.shape[2]
    return pl.pallas_call(
        _final_body,
        grid=(pl.cdiv(n, blk),),
        in_specs=[
            pl.BlockSpec((blk, d), lambda i: (i, 0)),
            pl.BlockSpec((NC, blk, d), lambda i: (0, i, 0)),
            pl.BlockSpec((NC, blk, ddeg), lambda i: (0, i, 0)),
        ],
        out_specs=pl.BlockSpec((blk, d), lambda i: (i, 0)),
        out_shape=jax.ShapeDtypeStruct((n, d), jnp.float32),
    )(hs, aggp, degp)


def _pick_layout(epw, d, npad, with_gather):
    for ch in (128, 120, 112, 104, 96, 88, 80, 72, 64, 56, 48, 40, 32, 24,
               16, 8):
        if epw % ch:
            continue
        steps = epw // ch
        for nb in (5, 4, 6, 3, 2):
            if steps % nb:
                continue
            words = npad * d + NS * (nb * ch * (d + 1)
                                     + (epw if with_gather else 0))
            if words <= 2_000_000:
                return ch, nb
    raise ValueError("no feasible SC layout")


@functools.lru_cache(maxsize=None)
def _make_agg(n, e, d, ones_table=False):
    nw = NC * NS
    epw = e // nw
    rpt = ((n + NS - 1) // NS + CPY - 1) // CPY * CPY
    npad = rpt * NS
    ch, nbuf = _pick_layout(epw, d, npad, not ones_table)
    steps = epw // ch
    cpy = CPY if (ch >= CPY and rpt % CPY == 0) else ch
    ncopy = rpt // cpy

    mesh = plsc.VectorSubcoreMesh(core_axis_name="c", subcore_axis_name="s",
                                  num_cores=NC, num_subcores=NS)

    scratch = [
        pltpu.VMEM_SHARED((npad, d), jnp.float32),
        [pltpu.VMEM((ch,), jnp.int32) for _ in range(nbuf)],
        [pltpu.VMEM((ch, d), jnp.float32) for _ in range(nbuf)],
        [pltpu.SemaphoreType.DMA for _ in range(nbuf)],
        [pltpu.SemaphoreType.DMA for _ in range(nbuf)],
    ]
    if not ones_table:
        scratch += [
            pltpu.VMEM((epw,), jnp.int32),
            [pltpu.SemaphoreType.DMA for _ in range(nbuf)],
        ]

    def body(table, srcv, dstv, zeros_nd, agg_out,
             acc, idx_d, rows, isem, ssem, srcall, gsem):
        c = lax.axis_index("c")
        s = lax.axis_index("s")
        wid = c * NS + s
        r0 = s * rpt
        base = wid * epw
        pltpu.sync_copy(zeros_nd.at[pl.ds(r0, rpt)], acc.at[pl.ds(r0, rpt)])
        if ones_table:
            for b in range(nbuf):
                pltpu.sync_copy(table.at[pl.ds(0, ch)], rows[b])
        else:
            pltpu.sync_copy(srcv.at[pl.ds(base, epw)], srcall)
        plsc.subcore_barrier()

        def super_step(g, carry):
            j0 = g * nbuf
            descs = []
            for b in range(nbuf):
                off = base + (j0 + b) * ch
                di = pltpu.async_copy(dstv.at[pl.ds(off, ch)], idx_d[b],
                                      isem[b])
                if ones_table:
                    descs.append((di, None))
                else:
                    gi = pltpu.async_copy(
                        table.at[srcall.at[pl.ds((j0 + b) * ch, ch)]],
                        rows[b], gsem[b])
                    descs.append((di, gi))
            sdescs = []
            for b, (di, gi) in enumerate(descs):
                di.wait()
                if gi is not None:
                    gi.wait()
                sdescs.append(pltpu.async_copy(rows[b], acc.at[idx_d[b]],
                                               ssem[b], add=True))
            for sd in sdescs:
                sd.wait()
            return carry

        lax.fori_loop(0, steps // nbuf, super_step, 0)
        plsc.subcore_barrier()
        pltpu.sync_copy(acc.at[pl.ds(r0, rpt)],
                        agg_out.at[c].at[pl.ds(r0, rpt)])

    params = (pltpu.CompilerParams(use_tc_tiling_on_sc=False)
              if d % 128 else None)
    out_type = [jax.ShapeDtypeStruct((NC, npad, d), jnp.float32)]
    if ones_table:
        def fn(table, dstv, zeros_nd, agg_out, acc, idx_d, rows, isem, ssem):
            body(table, None, dstv, zeros_nd, agg_out,
                 acc, idx_d, rows, isem, ssem, None, None)

        kern = functools.partial(pl.kernel, mesh=mesh, out_type=out_type,
                                 scratch_types=scratch,
                                 compiler_params=params)(fn)

        def run(table, src, dst):
            del src
            zeros_nd = jnp.zeros((npad, d), jnp.float32)
            return kern(table, dst, zeros_nd)[0]
    else:
        def fn(table, srcv, dstv, zeros_nd, agg_out, acc, idx_d, rows, isem,
               ssem, srcall, gsem):
            body(table, srcv, dstv, zeros_nd, agg_out,
                 acc, idx_d, rows, isem, ssem, srcall, gsem)

        kern = functools.partial(pl.kernel, mesh=mesh, out_type=out_type,
                                 scratch_types=scratch,
                                 compiler_params=params)(fn)

        def run(table, src, dst):
            zeros_nd = jnp.zeros((npad, d), jnp.float32)
            return kern(table, src, dst, zeros_nd)[0]

    return run


def kernel(x, edge_index, W_self0, W_neigh0, b0, W_self1, W_neigh1, b1,
           W_self2, W_neigh2, b2):
    src = edge_index[0]
    dst = edge_index[1]
    n, _ = x.shape
    e = src.shape[0]
    d_hid = W_self0.shape[1]

    agg_hid = _make_agg(n, e, d_hid)

    degp = _make_agg(n, e, d_hid, ones_table=True)(
        jnp.ones((n, d_hid), jnp.float32), src, dst)

    hs0, ht0 = _transform(x, W_self0, W_neigh0, b0)
    aggp0 = agg_hid(ht0, src, dst)
    hs1, ht1 = _combine_transform(hs0, aggp0, degp, W_self1, W_neigh1, b1)
    aggp1 = agg_hid(ht1, src, dst)
    hs2, ht2 = _combine_transform(hs1, aggp1, degp, W_self2, W_neigh2, b2)
    aggp2 = _make_agg(n, e, ht2.shape[1])(ht2, src, dst)
    return _final(hs2, aggp2, degp)
